# Initial kernel scaffold; baseline (speedup 1.0000x reference)
import jax
import jax.numpy as jnp
from jax.experimental import pallas as pl


def kernel(feats, edge_index, W_seq, b_seq, gin_W, gin_b, lin_W, lin_b, bn_g, bn_b, cls_W, cls_b):
    raise NotImplementedError("write your pallas kernel here")


# speedup vs baseline: 3.1483x; 3.1483x over previous
import jax
import jax.numpy as jnp
from jax import lax
from jax.experimental import pallas as pl
from jax.experimental.pallas import tpu as pltpu
from jax.experimental.pallas import tpu_sc as plsc

N = 50000
E = 800000
R = 64
ORDER = 3
FLOW_LEN = 100
BS = N // FLOW_LEN
C = 12

HALF = N // 2
NTILE = 16
EPT = E // NTILE
CHUNK = 128
NFULL = EPT // CHUNK
TAIL = EPT - NFULL * CHUNK
DUMMY_SPAN = 1024
ZROWS = 1627
ACC_ROWS = NTILE * ZROWS
WB = 1560

BLK = 2000
GRID = N // BLK
GROUPS = BLK // FLOW_LEN


def _segsum_body(h_hbm, src_hbm, dst_hbm, zeros_hbm, agg_hbm,
                 src_buf, dst_buf, dstl_buf, rows_buf,
                 src_t, dst_t, dstl_t, rows_t, acc, sem):
    cid = lax.axis_index("c")
    sid = lax.axis_index("s")
    core_base = cid * HALF

    pltpu.sync_copy(zeros_hbm, acc.at[pl.ds(sid * ZROWS, ZROWS)])
    plsc.subcore_barrier()

    e_base = sid * EPT

    def do_chunk(base, n, sbuf, dbuf, lbuf, rbuf):
        pltpu.sync_copy(src_hbm.at[pl.ds(base, n)], sbuf)
        pltpu.sync_copy(dst_hbm.at[pl.ds(base, n)], dbuf)
        for k in range(n // 16):
            dv = dbuf[pl.ds(16 * k, 16)]
            loc = dv - core_base
            ok = (loc >= 0) & (loc < HALF)
            alt = HALF + (dv & (DUMMY_SPAN - 1))
            lbuf[pl.ds(16 * k, 16)] = jnp.where(ok, loc, alt)
        pltpu.async_copy(h_hbm.at[sbuf], rbuf, sem).wait()
        pltpu.sync_copy(rbuf, acc.at[lbuf], add=True)

    def body(j, carry):
        do_chunk(e_base + j * CHUNK, CHUNK, src_buf, dst_buf, dstl_buf, rows_buf)
        return carry

    lax.fori_loop(0, NFULL, body, 0)
    do_chunk(e_base + NFULL * CHUNK, TAIL, src_t, dst_t, dstl_t, rows_t)

    plsc.subcore_barrier()

    @pl.when(sid < NTILE - 1)
    def _():
        pltpu.sync_copy(acc.at[pl.ds(sid * WB, WB)],
                        agg_hbm.at[pl.ds(core_base + sid * WB, WB)])

    @pl.when(sid == NTILE - 1)
    def _():
        rest = HALF - (NTILE - 1) * WB
        pltpu.sync_copy(acc.at[pl.ds((NTILE - 1) * WB, rest)],
                        agg_hbm.at[pl.ds(core_base + (NTILE - 1) * WB, rest)])


def _segsum(h, src, dst, zeros):
    return pl.kernel(
        _segsum_body,
        mesh=plsc.VectorSubcoreMesh(core_axis_name="c", subcore_axis_name="s"),
        compiler_params=pltpu.CompilerParams(use_tc_tiling_on_sc=False),
        out_type=jax.ShapeDtypeStruct((N, R), jnp.float32),
        scratch_types=[
            pltpu.VMEM((CHUNK,), jnp.int32),
            pltpu.VMEM((CHUNK,), jnp.int32),
            pltpu.VMEM((CHUNK,), jnp.int32),
            pltpu.VMEM((CHUNK, R), jnp.float32),
            pltpu.VMEM((TAIL,), jnp.int32),
            pltpu.VMEM((TAIL,), jnp.int32),
            pltpu.VMEM((TAIL,), jnp.int32),
            pltpu.VMEM((TAIL, R), jnp.float32),
            pltpu.VMEM_SHARED((ACC_ROWS, R), jnp.float32),
            pltpu.SemaphoreType.DMA,
        ],
    )(h, src, dst, zeros)


def _seed_body(f_ref, w_ref, b_ref, h_ref):
    h_ref[...] = f_ref[...] * w_ref[...] + b_ref[...]


def _pass1_body(h_ref, agg_ref, g_ref, gb_ref, l_ref, lb_ref, z_ref, s_ref, q_ref):
    x = h_ref[...] + agg_ref[...]
    z = jnp.dot(x, g_ref[...], preferred_element_type=jnp.float32) + gb_ref[...]
    z = jnp.dot(z, l_ref[...], preferred_element_type=jnp.float32) + lb_ref[...]
    z_ref[...] = z

    @pl.when(pl.program_id(0) == 0)
    def _():
        s_ref[...] = jnp.zeros_like(s_ref)
        q_ref[...] = jnp.zeros_like(q_ref)

    s_ref[...] += jnp.sum(z, axis=0, keepdims=True)
    q_ref[...] += jnp.sum(z * z, axis=0, keepdims=True)


def _stats_body(s_ref, q_ref, g_ref, b_ref, sc_ref, sh_ref):
    mean = s_ref[...] * (1.0 / N)
    var = q_ref[...] * (1.0 / N) - mean * mean
    inv = lax.rsqrt(var + 1e-5)
    scale = g_ref[...] * inv
    sc_ref[...] = scale
    sh_ref[...] = b_ref[...] - mean * scale


def _pass2_body(z_ref, sc_ref, sh_ref, h_ref, ro_ref):
    hn = jnp.maximum(z_ref[...] * sc_ref[...] + sh_ref[...], 0.0)
    h_ref[...] = hn
    ro_ref[...] = hn.reshape(GROUPS, FLOW_LEN, R).sum(axis=1)[None]


def _cls_body(r0_ref, r1_ref, r2_ref, w0_ref, w1_ref, w2_ref, b_ref, y_ref):
    y = jnp.dot(r0_ref[...], w0_ref[...], preferred_element_type=jnp.float32)
    y += jnp.dot(r1_ref[...], w1_ref[...], preferred_element_type=jnp.float32)
    y += jnp.dot(r2_ref[...], w2_ref[...], preferred_element_type=jnp.float32)
    y_ref[...] = y + b_ref[...]


def kernel(feats, edge_index, W_seq, b_seq, gin_W, gin_b, lin_W, lin_b, bn_g, bn_b, cls_W, cls_b):
    f32 = jnp.float32
    src = edge_index[0].astype(jnp.int32)
    dst = edge_index[1].astype(jnp.int32)
    zeros = jnp.zeros((ZROWS, R), f32)

    h = pl.pallas_call(
        _seed_body,
        grid=(GRID,),
        in_specs=[
            pl.BlockSpec((BLK, 1), lambda i: (i, 0)),
            pl.BlockSpec((1, R), lambda i: (0, 0)),
            pl.BlockSpec((1, R), lambda i: (0, 0)),
        ],
        out_specs=pl.BlockSpec((BLK, R), lambda i: (i, 0)),
        out_shape=jax.ShapeDtypeStruct((N, R), f32),
    )(feats.reshape(N, 1), W_seq, b_seq.reshape(1, R))

    ros = []
    for i in range(ORDER):
        agg = _segsum(h, src, dst, zeros)

        z, s, q = pl.pallas_call(
            _pass1_body,
            grid=(GRID,),
            in_specs=[
                pl.BlockSpec((BLK, R), lambda i: (i, 0)),
                pl.BlockSpec((BLK, R), lambda i: (i, 0)),
                pl.BlockSpec((R, R), lambda i: (0, 0)),
                pl.BlockSpec((1, R), lambda i: (0, 0)),
                pl.BlockSpec((R, R), lambda i: (0, 0)),
                pl.BlockSpec((1, R), lambda i: (0, 0)),
            ],
            out_specs=[
                pl.BlockSpec((BLK, R), lambda i: (i, 0)),
                pl.BlockSpec((1, R), lambda i: (0, 0)),
                pl.BlockSpec((1, R), lambda i: (0, 0)),
            ],
            out_shape=[
                jax.ShapeDtypeStruct((N, R), f32),
                jax.ShapeDtypeStruct((1, R), f32),
                jax.ShapeDtypeStruct((1, R), f32),
            ],
        )(h, agg, gin_W[i], gin_b[i].reshape(1, R), lin_W[i], lin_b[i].reshape(1, R))

        scale, shift = pl.pallas_call(
            _stats_body,
            out_shape=[
                jax.ShapeDtypeStruct((1, R), f32),
                jax.ShapeDtypeStruct((1, R), f32),
            ],
        )(s, q, bn_g[i].reshape(1, R), bn_b[i].reshape(1, R))

        h, ro = pl.pallas_call(
            _pass2_body,
            grid=(GRID,),
            in_specs=[
                pl.BlockSpec((BLK, R), lambda i: (i, 0)),
                pl.BlockSpec((1, R), lambda i: (0, 0)),
                pl.BlockSpec((1, R), lambda i: (0, 0)),
            ],
            out_specs=[
                pl.BlockSpec((BLK, R), lambda i: (i, 0)),
                pl.BlockSpec((1, GROUPS, R), lambda i: (i, 0, 0)),
            ],
            out_shape=[
                jax.ShapeDtypeStruct((N, R), f32),
                jax.ShapeDtypeStruct((GRID, GROUPS, R), f32),
            ],
        )(z, scale, shift)
        ros.append(ro.reshape(BS, R))

    wp = jnp.pad(cls_W, ((0, 0), (0, 128 - C)))
    bp = jnp.pad(cls_b, (0, 128 - C)).reshape(1, 128)
    y = pl.pallas_call(
        _cls_body,
        out_shape=jax.ShapeDtypeStruct((BS, 128), f32),
    )(ros[0], ros[1], ros[2], wp[0:R], wp[R:2 * R], wp[2 * R:3 * R], bp)
    return y[:, :C]


# speedup vs baseline: 6.0450x; 1.9201x over previous
import jax
import jax.numpy as jnp
from jax import lax
from jax.experimental import pallas as pl
from jax.experimental.pallas import tpu as pltpu
from jax.experimental.pallas import tpu_sc as plsc

N = 50000
E = 800000
R = 64
ORDER = 3
FLOW_LEN = 100
BS = N // FLOW_LEN
C = 12

HALF = N // 2
NTILE = 16
CHUNK = 128
EROWS = E // CHUNK
ROWS_PT = EROWS // NTILE
EXTRA = EROWS - ROWS_PT * NTILE
DUMMY_SPAN = 1024
ZROWS = 1627
ACC_ROWS = NTILE * ZROWS
WB = 1560

BLK = 2000
GRID = N // BLK
GROUPS = BLK // FLOW_LEN


def _segsum_body(h_hbm, src_hbm, dst_hbm, zeros_hbm, agg_hbm,
                 sb0, sb1, db0, db1, lb0, lb1, rb0, rb1, acc,
                 sg0, sg1, si0, si1):
    cid = lax.axis_index("c")
    sid = lax.axis_index("s")
    core_base = cid * HALF

    pltpu.sync_copy(zeros_hbm, acc.at[pl.ds(sid * ZROWS, ZROWS)])
    plsc.subcore_barrier()

    srcb = [sb0, sb1]
    dstb = [db0, db1]
    dstl = [lb0, lb1]
    rows = [rb0, rb1]
    sg = [sg0, sg1]
    si = [si0, si1]
    base_row = sid * ROWS_PT
    nloc = jnp.where(sid < EXTRA, ROWS_PT + 1, ROWS_PT)

    def row_of(j):
        return jnp.where(j < ROWS_PT, base_row + j, NTILE * ROWS_PT + sid)

    def sdesc(b, row):
        return pltpu.make_async_copy(src_hbm.at[row], srcb[b], si[b])

    def ddesc(b, row):
        return pltpu.make_async_copy(dst_hbm.at[row], dstb[b], si[b])

    def gdesc(b):
        return pltpu.make_async_copy(h_hbm.at[srcb[b]], rows[b], sg[b])

    def compute_dstl(b):
        for k in range(CHUNK // 16):
            dv = dstb[b][pl.ds(16 * k, 16)]
            loc = dv - core_base
            ok = (loc >= 0) & (loc < HALF)
            alt = HALF + (dv & (DUMMY_SPAN - 1))
            dstl[b][pl.ds(16 * k, 16)] = jnp.where(ok, loc, alt)

    pltpu.sync_copy(src_hbm.at[base_row], srcb[0])
    pltpu.sync_copy(dst_hbm.at[base_row], dstb[0])
    compute_dstl(0)
    gdesc(0).start()
    sdesc(1, row_of(1)).start()
    ddesc(1, row_of(1)).start()

    def step(j, b):
        bn = 1 - b
        nr = row_of(j + 1)

        @pl.when(j + 1 < nloc)
        def _():
            sdesc(bn, nr).wait()
            ddesc(bn, nr).wait()
            compute_dstl(bn)

        gdesc(b).wait()

        @pl.when(j + 1 < nloc)
        def _():
            gdesc(bn).start()

        @pl.when(j + 2 < nloc)
        def _():
            r2 = row_of(j + 2)
            sdesc(b, r2).start()
            ddesc(b, r2).start()

        pltpu.sync_copy(rows[b], acc.at[dstl[b]], add=True)

    def pair(jj, carry):
        j = jj * 2
        step(j, 0)
        step(j + 1, 1)
        return carry

    lax.fori_loop(0, ROWS_PT // 2, pair, 0)

    @pl.when(nloc > ROWS_PT)
    def _():
        gdesc(0).wait()
        pltpu.sync_copy(rows[0], acc.at[dstl[0]], add=True)

    plsc.subcore_barrier()

    @pl.when(sid < NTILE - 1)
    def _():
        pltpu.sync_copy(acc.at[pl.ds(sid * WB, WB)],
                        agg_hbm.at[pl.ds(core_base + sid * WB, WB)])

    @pl.when(sid == NTILE - 1)
    def _():
        rest = HALF - (NTILE - 1) * WB
        pltpu.sync_copy(acc.at[pl.ds((NTILE - 1) * WB, rest)],
                        agg_hbm.at[pl.ds(core_base + (NTILE - 1) * WB, rest)])


def _segsum(h, src, dst, zeros):
    return pl.kernel(
        _segsum_body,
        mesh=plsc.VectorSubcoreMesh(core_axis_name="c", subcore_axis_name="s"),
        compiler_params=pltpu.CompilerParams(use_tc_tiling_on_sc=False),
        out_type=jax.ShapeDtypeStruct((N, R), jnp.float32),
        scratch_types=[
            pltpu.VMEM((CHUNK,), jnp.int32),
            pltpu.VMEM((CHUNK,), jnp.int32),
            pltpu.VMEM((CHUNK,), jnp.int32),
            pltpu.VMEM((CHUNK,), jnp.int32),
            pltpu.VMEM((CHUNK,), jnp.int32),
            pltpu.VMEM((CHUNK,), jnp.int32),
            pltpu.VMEM((CHUNK, R), jnp.float32),
            pltpu.VMEM((CHUNK, R), jnp.float32),
            pltpu.VMEM_SHARED((ACC_ROWS, R), jnp.float32),
            pltpu.SemaphoreType.DMA,
            pltpu.SemaphoreType.DMA,
            pltpu.SemaphoreType.DMA,
            pltpu.SemaphoreType.DMA,
        ],
    )(h, src.reshape(EROWS, CHUNK), dst.reshape(EROWS, CHUNK), zeros)


def _seed_body(f_ref, w_ref, b_ref, h_ref):
    h_ref[...] = f_ref[...] * w_ref[...] + b_ref[...]


def _pass1_body(h_ref, agg_ref, g_ref, gb_ref, l_ref, lb_ref, z_ref, s_ref, q_ref):
    x = h_ref[...] + agg_ref[...]
    z = jnp.dot(x, g_ref[...], preferred_element_type=jnp.float32) + gb_ref[...]
    z = jnp.dot(z, l_ref[...], preferred_element_type=jnp.float32) + lb_ref[...]
    z_ref[...] = z

    @pl.when(pl.program_id(0) == 0)
    def _():
        s_ref[...] = jnp.zeros_like(s_ref)
        q_ref[...] = jnp.zeros_like(q_ref)

    s_ref[...] += jnp.sum(z, axis=0, keepdims=True)
    q_ref[...] += jnp.sum(z * z, axis=0, keepdims=True)


def _stats_body(s_ref, q_ref, g_ref, b_ref, sc_ref, sh_ref):
    mean = s_ref[...] * (1.0 / N)
    var = q_ref[...] * (1.0 / N) - mean * mean
    inv = lax.rsqrt(var + 1e-5)
    scale = g_ref[...] * inv
    sc_ref[...] = scale
    sh_ref[...] = b_ref[...] - mean * scale


def _pass2_body(z_ref, sc_ref, sh_ref, h_ref, ro_ref):
    hn = jnp.maximum(z_ref[...] * sc_ref[...] + sh_ref[...], 0.0)
    h_ref[...] = hn
    ro_ref[...] = hn.reshape(GROUPS, FLOW_LEN, R).sum(axis=1)[None]


def _cls_body(r0_ref, r1_ref, r2_ref, w0_ref, w1_ref, w2_ref, b_ref, y_ref):
    y = jnp.dot(r0_ref[...], w0_ref[...], preferred_element_type=jnp.float32)
    y += jnp.dot(r1_ref[...], w1_ref[...], preferred_element_type=jnp.float32)
    y += jnp.dot(r2_ref[...], w2_ref[...], preferred_element_type=jnp.float32)
    y_ref[...] = y + b_ref[...]


def kernel(feats, edge_index, W_seq, b_seq, gin_W, gin_b, lin_W, lin_b, bn_g, bn_b, cls_W, cls_b):
    f32 = jnp.float32
    src = edge_index[0].astype(jnp.int32)
    dst = edge_index[1].astype(jnp.int32)
    zeros = jnp.zeros((ZROWS, R), f32)

    h = pl.pallas_call(
        _seed_body,
        grid=(GRID,),
        in_specs=[
            pl.BlockSpec((BLK, 1), lambda i: (i, 0)),
            pl.BlockSpec((1, R), lambda i: (0, 0)),
            pl.BlockSpec((1, R), lambda i: (0, 0)),
        ],
        out_specs=pl.BlockSpec((BLK, R), lambda i: (i, 0)),
        out_shape=jax.ShapeDtypeStruct((N, R), f32),
    )(feats.reshape(N, 1), W_seq, b_seq.reshape(1, R))

    ros = []
    for i in range(ORDER):
        agg = _segsum(h, src, dst, zeros)

        z, s, q = pl.pallas_call(
            _pass1_body,
            grid=(GRID,),
            in_specs=[
                pl.BlockSpec((BLK, R), lambda i: (i, 0)),
                pl.BlockSpec((BLK, R), lambda i: (i, 0)),
                pl.BlockSpec((R, R), lambda i: (0, 0)),
                pl.BlockSpec((1, R), lambda i: (0, 0)),
                pl.BlockSpec((R, R), lambda i: (0, 0)),
                pl.BlockSpec((1, R), lambda i: (0, 0)),
            ],
            out_specs=[
                pl.BlockSpec((BLK, R), lambda i: (i, 0)),
                pl.BlockSpec((1, R), lambda i: (0, 0)),
                pl.BlockSpec((1, R), lambda i: (0, 0)),
            ],
            out_shape=[
                jax.ShapeDtypeStruct((N, R), f32),
                jax.ShapeDtypeStruct((1, R), f32),
                jax.ShapeDtypeStruct((1, R), f32),
            ],
        )(h, agg, gin_W[i], gin_b[i].reshape(1, R), lin_W[i], lin_b[i].reshape(1, R))

        scale, shift = pl.pallas_call(
            _stats_body,
            out_shape=[
                jax.ShapeDtypeStruct((1, R), f32),
                jax.ShapeDtypeStruct((1, R), f32),
            ],
        )(s, q, bn_g[i].reshape(1, R), bn_b[i].reshape(1, R))

        h, ro = pl.pallas_call(
            _pass2_body,
            grid=(GRID,),
            in_specs=[
                pl.BlockSpec((BLK, R), lambda i: (i, 0)),
                pl.BlockSpec((1, R), lambda i: (0, 0)),
                pl.BlockSpec((1, R), lambda i: (0, 0)),
            ],
            out_specs=[
                pl.BlockSpec((BLK, R), lambda i: (i, 0)),
                pl.BlockSpec((1, GROUPS, R), lambda i: (i, 0, 0)),
            ],
            out_shape=[
                jax.ShapeDtypeStruct((N, R), f32),
                jax.ShapeDtypeStruct((GRID, GROUPS, R), f32),
            ],
        )(z, scale, shift)
        ros.append(ro.reshape(BS, R))

    wp = jnp.pad(cls_W, ((0, 0), (0, 128 - C)))
    bp = jnp.pad(cls_b, (0, 128 - C)).reshape(1, 128)
    y = pl.pallas_call(
        _cls_body,
        out_shape=jax.ShapeDtypeStruct((BS, 128), f32),
    )(ros[0], ros[1], ros[2], wp[0:R], wp[R:2 * R], wp[2 * R:3 * R], bp)
    return y[:, :C]
